# baseline (speedup 1.0000x reference)
import functools
import math

import jax
import jax.numpy as jnp
from jax import lax
from jax.experimental import pallas as pl
from jax.experimental.pallas import tpu as pltpu
from jax.experimental.pallas import tpu_sc as plsc

EMBED_DIM = 64
SCALE = math.sqrt(EMBED_DIM)
LANES = 16

NUM_CORES = 2
NUM_SUBCORES = 16
NUM_WORKERS = NUM_CORES * NUM_SUBCORES

S_GROUPS = 8
C_GROUPS = 4
WIN = 128


def _emb_kernel(batch, seq):
    s_per_w = seq // S_GROUPS
    c_per_w = batch // C_GROUPS
    wins_per_row = c_per_w // WIN
    mesh = plsc.VectorSubcoreMesh(core_axis_name="c", subcore_axis_name="s")

    @functools.partial(
        pl.kernel,
        mesh=mesh,
        compiler_params=pltpu.CompilerParams(
            use_tc_tiling_on_sc=True, needs_layout_passes=False
        ),
        out_type=jax.ShapeDtypeStruct((seq, EMBED_DIM, batch), jnp.float32),
        scratch_types=[
            pltpu.VMEM((c_per_w,), jnp.int32),
            pltpu.VMEM((WIN, 128), jnp.float32),
            pltpu.VMEM((EMBED_DIM, WIN), jnp.float32),
            pltpu.SemaphoreType.DMA,
        ],
    )
    def k(xT_hbm, tab_hbm, out_hbm, idx_v, g_v, o_v, sem):
        wid = lax.axis_index("s") * NUM_CORES + lax.axis_index("c")
        sgrp = wid // C_GROUPS
        cgrp = wid % C_GROUPS
        s0 = sgrp * s_per_w
        b_base = cgrp * c_per_w

        lane_iota = lax.iota(jnp.int32, LANES)

        @pl.loop(0, s_per_w)
        def _(si):
            s = s0 + si
            pltpu.sync_copy(xT_hbm.at[s, pl.ds(b_base, c_per_w)], idx_v)

            @pl.loop(0, wins_per_row)
            def _(w):
                pltpu.async_copy(
                    tab_hbm.at[idx_v.at[pl.ds(w * WIN, WIN)]], g_v, sem
                ).wait()

                @pl.loop(0, EMBED_DIM)
                def _(r):
                    r16 = jnp.broadcast_to(r, (LANES,))
                    for blk in range(WIN // LANES):
                        rows = lane_iota + (blk * LANES)
                        vals = plsc.load_gather(g_v, [rows, r16])
                        o_v[r, pl.ds(blk * LANES, LANES)] = vals * SCALE

                pltpu.sync_copy(
                    o_v, out_hbm.at[s, :, pl.ds(b_base + w * WIN, WIN)]
                )

    return k


def kernel(x, table):
    b, s = x.shape
    v, d = table.shape
    tab128 = jnp.pad(table, ((0, 0), (0, 128 - d)))
    out3 = _emb_kernel(b, s)(x.T, tab128)
    return out3.transpose(2, 0, 1)


# speedup vs baseline: 2.8682x; 2.8682x over previous
import functools
import math

import jax
import jax.numpy as jnp
from jax import lax
from jax.experimental import pallas as pl
from jax.experimental.pallas import tpu as pltpu
from jax.experimental.pallas import tpu_sc as plsc

EMBED_DIM = 64
SCALE = math.sqrt(EMBED_DIM)
LANES = 16

NUM_CORES = 2
NUM_SUBCORES = 16
NUM_WORKERS = NUM_CORES * NUM_SUBCORES

S_GROUPS = 8
C_GROUPS = 4
WIN = 128


def _emb_kernel(batch, seq):
    s_per_w = seq // S_GROUPS
    c_per_w = batch // C_GROUPS
    wins_per_row = c_per_w // WIN
    n_win = s_per_w * wins_per_row
    mesh = plsc.VectorSubcoreMesh(core_axis_name="c", subcore_axis_name="s")

    @functools.partial(
        pl.kernel,
        mesh=mesh,
        compiler_params=pltpu.CompilerParams(
            use_tc_tiling_on_sc=True, needs_layout_passes=False
        ),
        out_type=jax.ShapeDtypeStruct((seq, EMBED_DIM, batch), jnp.float32),
        scratch_types=[
            pltpu.VMEM((s_per_w, c_per_w), jnp.int32),
            [pltpu.VMEM((WIN, 128), jnp.float32)] * 2,
            [pltpu.VMEM((EMBED_DIM, WIN), jnp.float32)] * 2,
            [pltpu.SemaphoreType.DMA] * 2,
            [pltpu.SemaphoreType.DMA] * 2,
        ],
    )
    def k(xT_hbm, tab_hbm, out_hbm, idx_v, g, o, gsem, osem):
        wid = lax.axis_index("s") * NUM_CORES + lax.axis_index("c")
        sgrp = wid // C_GROUPS
        cgrp = wid % C_GROUPS
        s0 = sgrp * s_per_w
        b_base = cgrp * c_per_w

        @pl.loop(0, s_per_w)
        def _(si):
            pltpu.sync_copy(
                xT_hbm.at[s0 + si, pl.ds(b_base, c_per_w)], idx_v.at[si]
            )

        rows16 = [
            lax.iota(jnp.int32, LANES) + (blk * LANES)
            for blk in range(WIN // LANES)
        ]

        def start_gather(win, b):
            si = win // wins_per_row
            bj = win % wins_per_row
            pltpu.async_copy(
                tab_hbm.at[idx_v.at[si, pl.ds(bj * WIN, WIN)]], g[b], gsem[b]
            )

        def wait_gather(b):
            pltpu.make_async_copy(
                tab_hbm.at[idx_v.at[0, pl.ds(0, WIN)]], g[b], gsem[b]
            ).wait()

        def transpose(b):
            @functools.partial(plsc.parallel_loop, 0, EMBED_DIM)
            def _(r):
                r16 = jnp.broadcast_to(r, (LANES,))
                for blk in range(WIN // LANES):
                    vals = plsc.load_gather(g[b], [rows16[blk], r16])
                    o[b][r, pl.ds(blk * LANES, LANES)] = vals * SCALE

        def start_out(win, b):
            s = s0 + win // wins_per_row
            c0 = b_base + (win % wins_per_row) * WIN
            pltpu.async_copy(o[b], out_hbm.at[s, :, pl.ds(c0, WIN)], osem[b])

        def wait_out(b):
            pltpu.make_async_copy(
                o[b], out_hbm.at[0, :, pl.ds(0, WIN)], osem[b]
            ).wait()

        start_gather(0, 0)
        start_gather(1, 1)

        for b in range(2):
            wait_gather(b)
            transpose(b)
            start_gather(2 + b, b)
            start_out(b, b)

        @pl.loop(1, n_win // 2 - 1)
        def _(t):
            for b in range(2):
                win = 2 * t + b
                wait_gather(b)
                wait_out(b)
                transpose(b)
                start_gather(win + 2, b)
                start_out(win, b)

        for b in range(2):
            win = n_win - 2 + b
            wait_gather(b)
            wait_out(b)
            transpose(b)
            start_out(win, b)

        for b in range(2):
            wait_out(b)

    return k


def kernel(x, table):
    b, s = x.shape
    v, d = table.shape
    tab128 = jnp.pad(table, ((0, 0), (0, 128 - d)))
    out3 = _emb_kernel(b, s)(x.T, tab128)
    return out3.transpose(2, 0, 1)


# speedup vs baseline: 3.1599x; 1.1017x over previous
import functools
import math

import jax
import jax.numpy as jnp
from jax import lax
from jax.experimental import pallas as pl
from jax.experimental.pallas import tpu as pltpu
from jax.experimental.pallas import tpu_sc as plsc

EMBED_DIM = 64
SCALE = math.sqrt(EMBED_DIM)
LANES = 16

NUM_CORES = 2
NUM_SUBCORES = 16
NUM_WORKERS = NUM_CORES * NUM_SUBCORES

S_GROUPS = 8
C_GROUPS = 4
WIN = 128


def _emb_kernel(batch, seq):
    s_per_w = seq // S_GROUPS
    c_per_w = batch // C_GROUPS
    wins_per_row = c_per_w // WIN
    n_win = s_per_w * wins_per_row
    mesh = plsc.VectorSubcoreMesh(core_axis_name="c", subcore_axis_name="s")

    @functools.partial(
        pl.kernel,
        mesh=mesh,
        compiler_params=pltpu.CompilerParams(
            use_tc_tiling_on_sc=True, needs_layout_passes=False
        ),
        out_type=jax.ShapeDtypeStruct((seq, EMBED_DIM, batch), jnp.float32),
        scratch_types=[
            pltpu.VMEM((s_per_w, c_per_w), jnp.int32),
            [pltpu.VMEM((WIN, 128), jnp.float32)] * 2,
            [pltpu.VMEM((EMBED_DIM, WIN), jnp.float32)] * 2,
            [pltpu.SemaphoreType.DMA] * 2,
            [pltpu.SemaphoreType.DMA] * 2,
        ],
    )
    def k(xT_hbm, tab_hbm, out_hbm, idx_v, g, o, gsem, osem):
        wid = lax.axis_index("s") * NUM_CORES + lax.axis_index("c")
        sgrp = wid // C_GROUPS
        cgrp = wid % C_GROUPS
        s0 = sgrp * s_per_w
        b_base = cgrp * c_per_w

        @pl.loop(0, s_per_w)
        def _(si):
            pltpu.sync_copy(
                xT_hbm.at[s0 + si, pl.ds(b_base, c_per_w)], idx_v.at[si]
            )

        rows16 = [
            lax.iota(jnp.int32, LANES) + (blk * LANES)
            for blk in range(WIN // LANES)
        ]

        def start_gather(win, b):
            si = win // wins_per_row
            bj = win % wins_per_row
            pltpu.async_copy(
                tab_hbm.at[idx_v.at[si, pl.ds(bj * WIN, WIN)]], g[b], gsem[b]
            )

        def wait_gather(b):
            pltpu.make_async_copy(
                tab_hbm.at[idx_v.at[0, pl.ds(0, WIN)]], g[b], gsem[b]
            ).wait()

        def transpose(b):
            @functools.partial(plsc.parallel_loop, 0, EMBED_DIM)
            def _(r):
                r16 = jnp.broadcast_to(r, (LANES,))
                for blk in range(WIN // LANES):
                    vals = plsc.load_gather(g[b], [rows16[blk], r16])
                    o[b][r, pl.ds(blk * LANES, LANES)] = vals

        def start_out(win, b):
            s = s0 + win // wins_per_row
            c0 = b_base + (win % wins_per_row) * WIN
            pltpu.async_copy(o[b], out_hbm.at[s, :, pl.ds(c0, WIN)], osem[b])

        def wait_out(b):
            pltpu.make_async_copy(
                o[b], out_hbm.at[0, :, pl.ds(0, WIN)], osem[b]
            ).wait()

        start_gather(0, 0)
        start_gather(1, 1)

        for b in range(2):
            wait_gather(b)
            transpose(b)
            start_gather(2 + b, b)
            start_out(b, b)

        @pl.loop(1, n_win // 2 - 1)
        def _(t):
            for b in range(2):
                win = 2 * t + b
                wait_gather(b)
                wait_out(b)
                transpose(b)
                start_gather(win + 2, b)
                start_out(win, b)

        for b in range(2):
            win = n_win - 2 + b
            wait_gather(b)
            wait_out(b)
            transpose(b)
            start_out(win, b)

        for b in range(2):
            wait_out(b)

    return k


_PREP_BLK = 2048


def _prep_kernel(v, d):
    grid = (v + _PREP_BLK - 1) // _PREP_BLK

    def body(in_ref, out_ref):
        t = jnp.transpose(in_ref[...], (1, 0)) * SCALE
        out_ref[...] = jnp.pad(t, ((0, 0), (0, 128 - d)))

    return pl.pallas_call(
        body,
        grid=(grid,),
        in_specs=[pl.BlockSpec((d, _PREP_BLK), lambda i: (0, i))],
        out_specs=pl.BlockSpec((_PREP_BLK, 128), lambda i: (i, 0)),
        out_shape=jax.ShapeDtypeStruct((v, 128), jnp.float32),
        compiler_params=pltpu.CompilerParams(
            dimension_semantics=("parallel",)
        ),
    )


def kernel(x, table):
    b, s = x.shape
    v, d = table.shape
    tab128 = _prep_kernel(v, d)(table.T)
    out3 = _emb_kernel(b, s)(x.T, tab128)
    return out3.transpose(2, 0, 1)


# speedup vs baseline: 4.3669x; 1.3820x over previous
import functools
import math

import jax
import jax.numpy as jnp
from jax import lax
from jax.experimental import pallas as pl
from jax.experimental.pallas import tpu as pltpu
from jax.experimental.pallas import tpu_sc as plsc

EMBED_DIM = 64
SCALE = math.sqrt(EMBED_DIM)
LANES = 16

NUM_CORES = 2
NUM_SUBCORES = 16
NUM_WORKERS = NUM_CORES * NUM_SUBCORES

S_GROUPS = 8
C_GROUPS = 4
WIN = 128


def _emb_kernel(batch, seq):
    s_per_w = seq // S_GROUPS
    c_per_w = batch // C_GROUPS
    wins_per_row = c_per_w // WIN
    n_win = s_per_w * wins_per_row
    mesh = plsc.VectorSubcoreMesh(core_axis_name="c", subcore_axis_name="s")

    @functools.partial(
        pl.kernel,
        mesh=mesh,
        compiler_params=pltpu.CompilerParams(
            use_tc_tiling_on_sc=True, needs_layout_passes=False
        ),
        out_type=jax.ShapeDtypeStruct((seq, EMBED_DIM, batch), jnp.float32),
        scratch_types=[
            pltpu.VMEM((s_per_w, c_per_w), jnp.int32),
            [pltpu.VMEM((WIN, 128), jnp.float32)] * 2,
            [pltpu.VMEM((EMBED_DIM, WIN), jnp.float32)] * 2,
            [pltpu.SemaphoreType.DMA] * 2,
            [pltpu.SemaphoreType.DMA] * 2,
        ],
    )
    def k(xT_hbm, tab_hbm, out_hbm, idx_v, g, o, gsem, osem):
        wid = lax.axis_index("s") * NUM_CORES + lax.axis_index("c")
        sgrp = wid // C_GROUPS
        cgrp = wid % C_GROUPS
        s0 = sgrp * s_per_w
        b_base = cgrp * c_per_w

        @pl.loop(0, s_per_w)
        def _(si):
            pltpu.sync_copy(
                xT_hbm.at[s0 + si, pl.ds(b_base, c_per_w)], idx_v.at[si]
            )

        rows16 = [
            lax.iota(jnp.int32, LANES) + (blk * LANES)
            for blk in range(WIN // LANES)
        ]

        def start_gather(win, b):
            si = win // wins_per_row
            bj = win % wins_per_row
            pltpu.async_copy(
                tab_hbm.at[idx_v.at[si, pl.ds(bj * WIN, WIN)]], g[b], gsem[b]
            )

        def wait_gather(b):
            pltpu.make_async_copy(
                tab_hbm.at[idx_v.at[0, pl.ds(0, WIN)]], g[b], gsem[b]
            ).wait()

        def transpose(b):
            @functools.partial(plsc.parallel_loop, 0, EMBED_DIM)
            def _(r):
                r16 = jnp.broadcast_to(r, (LANES,))
                for blk in range(WIN // LANES):
                    vals = plsc.load_gather(g[b], [rows16[blk], r16])
                    o[b][r, pl.ds(blk * LANES, LANES)] = vals

        def start_out(win, b):
            s = s0 + win // wins_per_row
            c0 = b_base + (win % wins_per_row) * WIN
            pltpu.async_copy(o[b], out_hbm.at[s, :, pl.ds(c0, WIN)], osem[b])

        def wait_out(b):
            pltpu.make_async_copy(
                o[b], out_hbm.at[0, :, pl.ds(0, WIN)], osem[b]
            ).wait()

        start_gather(0, 0)
        start_gather(1, 1)

        for b in range(2):
            wait_gather(b)
            transpose(b)
            start_gather(2 + b, b)
            start_out(b, b)

        @pl.loop(1, n_win // 2 - 1)
        def _(t):
            for b in range(2):
                win = 2 * t + b
                wait_gather(b)
                wait_out(b)
                transpose(b)
                start_gather(win + 2, b)
                start_out(win, b)

        for b in range(2):
            win = n_win - 2 + b
            wait_gather(b)
            wait_out(b)
            transpose(b)
            start_out(win, b)

        for b in range(2):
            wait_out(b)

    return k


_PREP_BLK = 8192


def _prep_kernel(v, d):
    grid = (v + _PREP_BLK - 1) // _PREP_BLK

    def body(in_ref, out_ref):
        t = jnp.transpose(in_ref[...], (1, 0)) * SCALE
        out_ref[...] = jnp.pad(t, ((0, 0), (0, 128 - d)))

    return pl.pallas_call(
        body,
        grid=(grid,),
        in_specs=[pl.BlockSpec((d, _PREP_BLK), lambda i: (0, i))],
        out_specs=pl.BlockSpec((_PREP_BLK, 128), lambda i: (i, 0)),
        out_shape=jax.ShapeDtypeStruct((v, 128), jnp.float32),
        compiler_params=pltpu.CompilerParams(
            dimension_semantics=("parallel",)
        ),
    )


def kernel(x, table):
    b, s = x.shape
    v, d = table.shape
    tab128 = _prep_kernel(v, d)(table.T)
    out3 = _emb_kernel(b, s)(x.T, tab128)
    return out3.transpose(2, 0, 1)
